# baseline (speedup 1.0000x reference)
import functools

import jax
import jax.numpy as jnp
from jax import lax
from jax.experimental import pallas as pl
from jax.experimental.pallas import tpu as pltpu
from jax.experimental.pallas import tpu_sc as plsc


def kernel(x, queue):
    del queue
    B, D = x.shape
    info = plsc.get_sparse_core_info()
    nw = info.num_cores * info.num_subcores
    rows_per_w = B // nw

    mesh = plsc.VectorSubcoreMesh(core_axis_name="c", subcore_axis_name="s")

    @functools.partial(
        pl.kernel,
        mesh=mesh,
        out_type=jax.ShapeDtypeStruct((B, D), x.dtype),
    )
    def copy_rows(x_hbm, out_hbm):
        wid = lax.axis_index("s") * info.num_cores + lax.axis_index("c")
        base = wid * rows_per_w
        pltpu.sync_copy(
            x_hbm.at[pl.ds(base, rows_per_w)],
            out_hbm.at[pl.ds(base, rows_per_w)],
        )

    return copy_rows(x)


# speedup vs baseline: 1.2849x; 1.2849x over previous
import jax
import jax.numpy as jnp
from jax.experimental import pallas as pl
from jax.experimental.pallas import tpu as pltpu


def kernel(x, queue):
    del queue

    def body(x_hbm, o_hbm, sem):
        pltpu.make_async_copy(x_hbm, o_hbm, sem).start()
        pltpu.make_async_copy(x_hbm, o_hbm, sem).wait()

    return pl.pallas_call(
        body,
        in_specs=[pl.BlockSpec(memory_space=pl.ANY)],
        out_specs=pl.BlockSpec(memory_space=pl.ANY),
        out_shape=jax.ShapeDtypeStruct(x.shape, x.dtype),
        scratch_shapes=[pltpu.SemaphoreType.DMA],
    )(x)


# speedup vs baseline: 14.0321x; 10.9206x over previous
import jax
import jax.numpy as jnp
from jax.experimental import pallas as pl
from jax.experimental.pallas import tpu as pltpu

_ROWS_BLK = 512


def kernel(x, queue):
    del queue
    B, D = x.shape

    def body(x_ref, o_ref):
        o_ref[...] = x_ref[...]

    return pl.pallas_call(
        body,
        grid=(B // _ROWS_BLK,),
        in_specs=[pl.BlockSpec((_ROWS_BLK, D), lambda i: (i, 0))],
        out_specs=pl.BlockSpec((_ROWS_BLK, D), lambda i: (i, 0)),
        out_shape=jax.ShapeDtypeStruct((B, D), x.dtype),
    )(x)


# speedup vs baseline: 31.5653x; 2.2495x over previous
import jax
import jax.numpy as jnp
from jax.experimental import pallas as pl
from jax.experimental.pallas import tpu as pltpu

_N_CHUNKS = 8


def kernel(x, queue):
    del queue
    B, D = x.shape
    ch = B // _N_CHUNKS

    def body(x_hbm, o_hbm, buf, in_sems, out_sems):
        reads = [
            pltpu.make_async_copy(
                x_hbm.at[pl.ds(i * ch, ch)], buf.at[pl.ds(i * ch, ch)],
                in_sems.at[i])
            for i in range(_N_CHUNKS)
        ]
        writes = [
            pltpu.make_async_copy(
                buf.at[pl.ds(i * ch, ch)], o_hbm.at[pl.ds(i * ch, ch)],
                out_sems.at[i])
            for i in range(_N_CHUNKS)
        ]
        for r in reads:
            r.start()
        for r, w in zip(reads, writes):
            r.wait()
            w.start()
        for w in writes:
            w.wait()

    return pl.pallas_call(
        body,
        in_specs=[pl.BlockSpec(memory_space=pl.ANY)],
        out_specs=pl.BlockSpec(memory_space=pl.ANY),
        out_shape=jax.ShapeDtypeStruct((B, D), x.dtype),
        scratch_shapes=[
            pltpu.VMEM((B, D), x.dtype),
            pltpu.SemaphoreType.DMA((_N_CHUNKS,)),
            pltpu.SemaphoreType.DMA((_N_CHUNKS,)),
        ],
    )(x)
